# Initial kernel scaffold; baseline (speedup 1.0000x reference)
import jax
import jax.numpy as jnp
from jax.experimental import pallas as pl


def kernel(x, edge_index, edge_weight, W0, S0, b0, g0, be0, W1, S1, b1, g1, be1, W2, S2, b2, g2, be2):
    raise NotImplementedError("write your pallas kernel here")


# speedup vs baseline: 3.4858x; 3.4858x over previous
import functools

import jax
import jax.numpy as jnp
from jax import lax
from jax.experimental import pallas as pl
from jax.experimental.pallas import tpu as pltpu
from jax.experimental.pallas import tpu_sc as plsc

N = 10000
D = 128
E = 320000
NC = 2
NS = 16
NW = NC * NS
EPW = E // NW
CHUNK = 80
NCHUNKS = EPW // CHUNK
NP = 10240
RPW = NP // NS
_ZCOPIES = RPW // CHUNK


def _spmm_body(x_hbm, src_hbm, dst_hbm, ew_hbm, out_hbm,
               src_v, dst_v, ew_v, rows_v, acc_sh, gsem):
    c = lax.axis_index("c")
    s = lax.axis_index("s")

    def zrow(k, carry):
        for j in range(D // 16):
            rows_v[k, pl.ds(j * 16, 16)] = jnp.zeros((16,), jnp.float32)
        return carry
    lax.fori_loop(0, CHUNK, zrow, 0)
    row0 = s * RPW
    for t in range(_ZCOPIES):
        pltpu.sync_copy(rows_v, acc_sh.at[pl.ds(row0 + t * CHUNK, CHUNK)])
    plsc.subcore_barrier()

    ebase = (s * NC + c) * EPW

    def body(i, carry):
        base = ebase + i * CHUNK
        pltpu.sync_copy(src_hbm.at[pl.ds(base, CHUNK)], src_v)
        pltpu.sync_copy(dst_hbm.at[pl.ds(base, CHUNK)], dst_v)
        pltpu.sync_copy(ew_hbm.at[pl.ds(base, CHUNK)], ew_v)
        pltpu.async_copy(x_hbm.at[src_v], rows_v, gsem).wait()

        def scale(k, c2):
            w = plsc.load_gather(ew_v, [jnp.zeros((16,), jnp.int32) + k])
            for j in range(D // 16):
                rows_v[k, pl.ds(j * 16, 16)] = rows_v[k, pl.ds(j * 16, 16)] * w
            return c2
        lax.fori_loop(0, CHUNK, scale, 0)
        pltpu.sync_copy(rows_v, acc_sh.at[dst_v], add=True)
        return carry
    lax.fori_loop(0, NCHUNKS, body, 0)
    plsc.subcore_barrier()

    pltpu.sync_copy(acc_sh.at[pl.ds(row0, RPW)],
                    out_hbm.at[c, pl.ds(row0, RPW)])


@jax.jit
def _spmm(x, src, dst, ew):
    mesh = plsc.VectorSubcoreMesh(core_axis_name="c", subcore_axis_name="s")
    return pl.kernel(
        _spmm_body,
        out_type=jax.ShapeDtypeStruct((NC, NP, D), jnp.float32),
        mesh=mesh,
        scratch_types=[
            pltpu.VMEM((CHUNK,), jnp.int32),
            pltpu.VMEM((CHUNK,), jnp.int32),
            pltpu.VMEM((CHUNK,), jnp.float32),
            pltpu.VMEM((CHUNK, D), jnp.float32),
            pltpu.VMEM_SHARED((NP, D), jnp.float32),
            pltpu.SemaphoreType.DMA,
        ],
        compiler_params=pltpu.CompilerParams(needs_layout_passes=False),
    )(x, src, dst, ew)


def _dense_body(z_ref, xin_ref, W_ref, S_ref, b_ref, g_ref, be_ref, out_ref):
    z = z_ref[0, :N, :] + z_ref[1, :N, :]
    xin = xin_ref[...]
    h = jnp.dot(z, W_ref[...], preferred_element_type=jnp.float32)
    h = h + jnp.dot(xin, S_ref[...], preferred_element_type=jnp.float32)
    h = h + b_ref[...]
    mean = jnp.mean(h, axis=0, keepdims=True)
    cen = h - mean
    var = jnp.mean(cen * cen, axis=0, keepdims=True)
    hn = cen * lax.rsqrt(var + 1e-5) * g_ref[...] + be_ref[...]
    out_ref[...] = xin + jnp.maximum(hn, 0.0)


def _dense(z, xin, W, S, b, g, be):
    return pl.pallas_call(
        _dense_body,
        out_shape=jax.ShapeDtypeStruct((N, D), jnp.float32),
    )(z, xin, W, S, b.reshape(1, D), g.reshape(1, D), be.reshape(1, D))


def kernel(x, edge_index, edge_weight,
           W0, S0, b0, g0, be0,
           W1, S1, b1, g1, be1,
           W2, S2, b2, g2, be2):
    src = edge_index[0].astype(jnp.int32)
    dst = edge_index[1].astype(jnp.int32)
    ew = edge_weight.astype(jnp.float32)
    z0 = _spmm(x, src, dst, ew)
    x1 = _dense(z0, x, W0, S0, b0, g0, be0)
    z2 = _spmm(x1, src, dst, ew)
    return _dense(z2, x1, W2, S2, b2, g2, be2)
